# Initial kernel scaffold; baseline (speedup 1.0000x reference)
import jax
import jax.numpy as jnp
from jax.experimental import pallas as pl


def kernel(x, edge_index, edge_weight, sparse_w, sparse_b, bn_s_g, bn_s_b, W1, b1, bn1_g, bn1_b, W2, b2, bn2_g, bn2_b, lin_W, lin_b):
    raise NotImplementedError("write your pallas kernel here")


# speedup vs baseline: 1.2897x; 1.2897x over previous
import jax
import jax.numpy as jnp
import numpy as np
from jax.experimental import pallas as pl
from jax.experimental.pallas import tpu as pltpu

N = 50000
E = 800000
IN_F = 128
OUT_MASK_F = 128
EPS = 1e-5

_ROW_BLK = 2000


def _stage_a_body(x_ref, wa_ref, wb_ref, b_ref, o_ref):
    xb = x_ref[...]
    xr = jnp.roll(xb, 1, axis=1)
    o_ref[...] = jnp.maximum(xb * wa_ref[...] + xr * wb_ref[...] + b_ref[...], 0.0)


def _stage_a(x, sparse_w, sparse_b):
    wa = sparse_w[:IN_F].reshape(1, IN_F)
    wb = jnp.roll(sparse_w[IN_F:], 1).reshape(1, IN_F)
    b = sparse_b.reshape(1, IN_F)
    grid = (N // _ROW_BLK,)
    return pl.pallas_call(
        _stage_a_body,
        grid=grid,
        in_specs=[
            pl.BlockSpec((_ROW_BLK, IN_F), lambda i: (i, 0)),
            pl.BlockSpec((1, IN_F), lambda i: (0, 0)),
            pl.BlockSpec((1, IN_F), lambda i: (0, 0)),
            pl.BlockSpec((1, IN_F), lambda i: (0, 0)),
        ],
        out_specs=pl.BlockSpec((_ROW_BLK, IN_F), lambda i: (i, 0)),
        out_shape=jax.ShapeDtypeStruct((N, IN_F), jnp.float32),
    )(x, wa, wb, b)


def _batchnorm(x, g, b):
    mu = jnp.mean(x, axis=0)
    var = jnp.var(x, axis=0)
    return (x - mu) / jnp.sqrt(var + EPS) * g[None, :] + b[None, :]


def _gcn_conv(x, row, col, edge_weight, dinv, W, b):
    norm = dinv[row] * edge_weight * dinv[col]
    h = x @ W
    msg = h[row] * norm[:, None]
    out = jax.ops.segment_sum(msg, col, num_segments=N)
    out = out + h * (dinv * dinv)[:, None]
    return out + b[None, :]


def kernel(x, edge_index, edge_weight, sparse_w, sparse_b, bn_s_g, bn_s_b,
           W1, b1, bn1_g, bn1_b, W2, b2, bn2_g, bn2_b, lin_W, lin_b):
    h = _stage_a(x, sparse_w, sparse_b)
    h = _batchnorm(h, bn_s_g, bn_s_b)
    row = edge_index[0]
    col = edge_index[1]
    deg = jax.ops.segment_sum(edge_weight, col, num_segments=N) + 1.0
    dinv = jax.lax.rsqrt(deg)
    h = jax.nn.relu(_gcn_conv(h, row, col, edge_weight, dinv, W1, b1))
    h = _batchnorm(h, bn1_g, bn1_b)
    h = jax.nn.relu(_gcn_conv(h, row, col, edge_weight, dinv, W2, b2))
    h = _batchnorm(h, bn2_g, bn2_b)
    return h @ lin_W + lin_b[None, :]


# speedup vs baseline: 3.1255x; 2.4235x over previous
import functools

import jax
import jax.numpy as jnp
from jax import lax
from jax.experimental import pallas as pl
from jax.experimental.pallas import tpu as pltpu
from jax.experimental.pallas import tpu_sc as plsc

N = 50000
E = 800000
IN_F = 128
EPS = 1e-5

_ROW_BLK = 2000
_CH = 64
_EP = 802816
_DEG_PAD = 51200


def _stage_a_body(x_ref, wa_ref, wb_ref, b_ref, o_ref):
    xb = x_ref[...]
    xr = jnp.roll(xb, 1, axis=1)
    o_ref[...] = jnp.maximum(xb * wa_ref[...] + xr * wb_ref[...] + b_ref[...], 0.0)


def _stage_a(x, sparse_w, sparse_b):
    wa = sparse_w[:IN_F].reshape(1, IN_F)
    wb = jnp.roll(sparse_w[IN_F:], 1).reshape(1, IN_F)
    b = sparse_b.reshape(1, IN_F)
    return pl.pallas_call(
        _stage_a_body,
        grid=(N // _ROW_BLK,),
        in_specs=[
            pl.BlockSpec((_ROW_BLK, IN_F), lambda i: (i, 0)),
            pl.BlockSpec((1, IN_F), lambda i: (0, 0)),
            pl.BlockSpec((1, IN_F), lambda i: (0, 0)),
            pl.BlockSpec((1, IN_F), lambda i: (0, 0)),
        ],
        out_specs=pl.BlockSpec((_ROW_BLK, IN_F), lambda i: (i, 0)),
        out_shape=jax.ShapeDtypeStruct((N, IN_F), jnp.float32),
    )(x, wa, wb, b)


_MESH = plsc.VectorSubcoreMesh(core_axis_name="c", subcore_axis_name="s")


def _sc_degree(colp, ewp):
    nchunk = _EP // (32 * _CH)

    @functools.partial(
        pl.kernel,
        out_type=jax.ShapeDtypeStruct((2, _DEG_PAD), jnp.float32),
        mesh=_MESH,
        scratch_types=[
            pltpu.VMEM((_CH,), jnp.int32),
            pltpu.VMEM((_CH,), jnp.float32),
            pltpu.VMEM((3200,), jnp.float32),
            pltpu.VMEM_SHARED((_DEG_PAD,), jnp.float32),
        ],
    )
    def k(col_hbm, ew_hbm, out_hbm, colv, ewv, zv, deg_sp):
        c = lax.axis_index("c")
        s = lax.axis_index("s")
        wid = c * 16 + s

        @pl.loop(0, 3200 // 16)
        def _(i):
            zv[pl.ds(i * 16, 16)] = jnp.zeros((16,), jnp.float32)

        pltpu.sync_copy(zv, deg_sp.at[pl.ds(s * 3200, 3200)])
        plsc.subcore_barrier()

        @pl.loop(0, nchunk)
        def _(kk):
            base = (wid * nchunk + kk) * _CH
            pltpu.sync_copy(col_hbm.at[pl.ds(base, _CH)], colv)
            pltpu.sync_copy(ew_hbm.at[pl.ds(base, _CH)], ewv)
            pltpu.sync_copy(ewv, deg_sp.at[colv], add=True)

        plsc.subcore_barrier()

        @pl.when(s == 0)
        def _():
            pltpu.sync_copy(deg_sp, out_hbm.at[c])

    return k(colp, ewp)


def _batchnorm(x, g, b):
    mu = jnp.mean(x, axis=0)
    var = jnp.var(x, axis=0)
    return (x - mu) / jnp.sqrt(var + EPS) * g[None, :] + b[None, :]


def _conv_layer(h, W, b, dinv, rowp, colp, ewp):
    g = (h @ W) * dinv[:, None]
    msg = g[rowp[:E]] * ewp[:E, None]
    conv = jax.ops.segment_sum(msg, colp[:E], num_segments=N)
    return dinv[:, None] * (conv + g) + b[None, :]


def kernel(x, edge_index, edge_weight, sparse_w, sparse_b, bn_s_g, bn_s_b,
           W1, b1, bn1_g, bn1_b, W2, b2, bn2_g, bn2_b, lin_W, lin_b):
    pad = _EP - E
    rowp = jnp.concatenate([edge_index[0], jnp.zeros((pad,), jnp.int32)])
    colp = jnp.concatenate([edge_index[1], jnp.zeros((pad,), jnp.int32)])
    ewp = jnp.concatenate([edge_weight, jnp.zeros((pad,), jnp.float32)])

    degp = _sc_degree(colp, ewp)
    deg = degp[0, :N] + degp[1, :N] + 1.0
    dinv = lax.rsqrt(deg)

    h = _stage_a(x, sparse_w, sparse_b)
    h = _batchnorm(h, bn_s_g, bn_s_b)
    h = jax.nn.relu(_conv_layer(h, W1, b1, dinv, rowp, colp, ewp))
    h = _batchnorm(h, bn1_g, bn1_b)
    h = jax.nn.relu(_conv_layer(h, W2, b2, dinv, rowp, colp, ewp))
    h = _batchnorm(h, bn2_g, bn2_b)
    return h @ lin_W + lin_b[None, :]
